# baseline (speedup 1.0000x reference)
import numpy as np
import jax
import jax.numpy as jnp
from jax import lax
from jax.experimental import pallas as pl
from jax.experimental.pallas import tpu as pltpu
from jax.experimental.pallas import tpu_sc as plsc

_ISIZE = 224
_PSIZE = 30
_rng = np.random.RandomState(0)
_X = int(_rng.randint(0, _ISIZE - _PSIZE))
_Y = int(_rng.randint(0, _ISIZE - _PSIZE))

_IMG = 3 * _ISIZE * _ISIZE
_IMGR = _IMG // 128
_SEGR = 168
_SEG = _SEGR * 128
_NSEG = _IMGR // _SEGR

_row_spans = {}
for _c in range(3):
    for _r in range(_PSIZE):
        _f0 = _c * _ISIZE * _ISIZE + (_X + _r) * _ISIZE + _Y
        _seg = _f0 // _SEG
        assert (_f0 + _PSIZE - 1) // _SEG == _seg
        _row_spans.setdefault(_seg, []).append((_c, _r, _f0))

_PIECES = []
for _seg in sorted(_row_spans):
    _rows = _row_spans[_seg]
    _lo = min(f for _, _, f in _rows) - _seg * _SEG
    _hi = max(f for _, _, f in _rows) + _PSIZE - _seg * _SEG
    _d0 = _lo // 128
    _n = -(-(_hi - _d0 * 128) // 128)
    _n = -(-_n // 16) * 16
    _PIECES.append((_seg, _d0, _n,
                    [(f - _seg * _SEG - _d0 * 128, c, r) for c, r, f in _rows]))
_NP = len(_PIECES)
_CRROWS = max(n for _, _, n, _ in _PIECES)
_BUFR = max(_SEGR, max(d0 + n for _, d0, n, _ in _PIECES))
_IDX_LENS = sorted({n for _, _, n, _ in _PIECES})
_SEG2PIECE = {seg: k for k, (seg, _, _, _) in enumerate(_PIECES)}
_RING = 5
_PRIME = 3
_OUTLAG = 2


def _cruns_kernel(p_ref, cr_ref):
    cr_ref[...] = jnp.zeros_like(cr_ref)
    for k, (_, _, _, rows) in enumerate(_PIECES):
        for off, c, r in rows:
            cr_ref[k, off:off + _PSIZE] = p_ref[0, c, r, :]


def _sc_add_body(x_ref, cr_ref, ar_refs, o_ref, bufs, idx_bufs, csem,
                 in_sems, out_sems, add_sem):
    info = plsc.get_sparse_core_info()
    nw = info.num_cores * info.num_subcores
    wid = lax.axis_index("s") * info.num_cores + lax.axis_index("c")
    n_imgs = x_ref.shape[0] // _IMGR // nw
    base = wid * (n_imgs * _IMGR)

    for i in range(len(_IDX_LENS)):
        pltpu.make_async_copy(ar_refs[i], idx_bufs[i], csem).start()
        pltpu.make_async_copy(ar_refs[i], idx_bufs[i], csem).wait()

    n = n_imgs * _NSEG

    def rows_of(c):
        img, seg = divmod(c, _NSEG)
        return base + img * _IMGR + seg * _SEGR

    def in_copy(c, b):
        return pltpu.make_async_copy(
            x_ref.at[pl.ds(rows_of(c), _SEGR)],
            bufs.at[b, pl.ds(0, _SEGR)], in_sems.at[b])

    def out_copy(c, b):
        return pltpu.make_async_copy(
            bufs.at[b, pl.ds(0, _SEGR)],
            o_ref.at[pl.ds(rows_of(c), _SEGR)], out_sems.at[b])

    for c in range(min(_PRIME, n)):
        in_copy(c, c % _RING).start()
    for c in range(n):
        b = c % _RING
        if c >= _OUTLAG:
            out_copy(c - _OUTLAG, (c - _OUTLAG) % _RING).wait()
        if c + _PRIME < n:
            in_copy(c + _PRIME, (c + _PRIME) % _RING).start()
        in_copy(c, b).wait()
        seg = c % _NSEG
        k = _SEG2PIECE.get(seg)
        if k is not None:
            _, d0, nrows, _ = _PIECES[k]
            i = _IDX_LENS.index(nrows)
            add = pltpu.make_async_copy(
                cr_ref.at[k].at[idx_bufs[i]],
                bufs.at[b, pl.ds(d0, nrows)], add_sem)
            add.start(add=True)
            add.wait()
        out_copy(c, b).start()
    for c in range(max(0, n - _OUTLAG), n):
        out_copy(c, c % _RING).wait()


def kernel(x, patch):
    B = x.shape[0]
    cruns = pl.pallas_call(
        _cruns_kernel,
        out_shape=jax.ShapeDtypeStruct((_NP, _CRROWS * 128), x.dtype),
    )(patch)
    cr3 = cruns.reshape(_NP, _CRROWS, 128)
    x3 = x.reshape(B * _IMGR, 128)
    aranges = [jnp.arange(ln, dtype=jnp.int32) for ln in _IDX_LENS]

    mesh = plsc.VectorSubcoreMesh(core_axis_name="c", subcore_axis_name="s")
    out = pl.kernel(
        _sc_add_body,
        out_type=jax.ShapeDtypeStruct((B * _IMGR, 128), x.dtype),
        mesh=mesh,
        scratch_types=[
            pltpu.VMEM((_RING, _BUFR, 128), x.dtype),
            [pltpu.VMEM((ln,), jnp.int32) for ln in _IDX_LENS],
            pltpu.SemaphoreType.DMA,
            pltpu.SemaphoreType.DMA((_RING,)),
            pltpu.SemaphoreType.DMA((_RING,)),
            pltpu.SemaphoreType.DMA,
        ],
    )(x3, cr3, aranges)
    return out.reshape(x.shape)


# speedup vs baseline: 1.0042x; 1.0042x over previous
import numpy as np
import jax
import jax.numpy as jnp
from jax import lax
from jax.experimental import pallas as pl
from jax.experimental.pallas import tpu as pltpu
from jax.experimental.pallas import tpu_sc as plsc

_ISIZE = 224
_PSIZE = 30
_rng = np.random.RandomState(0)
_X = int(_rng.randint(0, _ISIZE - _PSIZE))
_Y = int(_rng.randint(0, _ISIZE - _PSIZE))

_IMG = 3 * _ISIZE * _ISIZE
_TILE = 8 * 128
_IMGT = _IMG // _TILE
_SEGT = 21
_SEG = _SEGT * _TILE
_NSEG = _IMGT // _SEGT

_row_spans = {}
for _c in range(3):
    for _r in range(_PSIZE):
        _f0 = _c * _ISIZE * _ISIZE + (_X + _r) * _ISIZE + _Y
        _seg = _f0 // _SEG
        assert (_f0 + _PSIZE - 1) // _SEG == _seg
        _row_spans.setdefault(_seg, []).append((_c, _r, _f0))

_PIECES = []
for _seg in sorted(_row_spans):
    _rows = _row_spans[_seg]
    _lo = min(f for _, _, f in _rows) - _seg * _SEG
    _hi = max(f for _, _, f in _rows) + _PSIZE - _seg * _SEG
    _t0 = _lo // _TILE
    _nt = -(-(_hi - _t0 * _TILE) // _TILE)
    _PIECES.append((_seg, _t0, _nt,
                    [(f - _seg * _SEG - _t0 * _TILE, c, r) for c, r, f in _rows]))
_NP = len(_PIECES)
_CRT = max(nt for _, _, nt, _ in _PIECES)
_BUFT = max(_SEGT, max(t0 + nt for _, t0, nt, _ in _PIECES))
_SEG2PIECE = {seg: k for k, (seg, _, _, _) in enumerate(_PIECES)}
_RING = 5
_PRIME = 3
_OUTLAG = 2


def _cruns_kernel(p_ref, cr_ref):
    cr_ref[...] = jnp.zeros_like(cr_ref)
    for k, (_, _, _, rows) in enumerate(_PIECES):
        for off, c, r in rows:
            cr_ref[k, off:off + _PSIZE] = p_ref[0, c, r, :]


def _sc_add_body(x_ref, cr_ref, ar_ref, o_ref, bufs, idx_buf, csem,
                 in_sems, out_sems, add_sem):
    info = plsc.get_sparse_core_info()
    nw = info.num_cores * info.num_subcores
    wid = lax.axis_index("s") * info.num_cores + lax.axis_index("c")
    n_imgs = x_ref.shape[0] // _IMGT // nw
    base = wid * (n_imgs * _IMGT)

    pltpu.make_async_copy(ar_ref, idx_buf, csem).start()
    pltpu.make_async_copy(ar_ref, idx_buf, csem).wait()

    n = n_imgs * _NSEG

    def tiles_of(c):
        img, seg = divmod(c, _NSEG)
        return base + img * _IMGT + seg * _SEGT

    def in_copy(c, b):
        return pltpu.make_async_copy(
            x_ref.at[pl.ds(tiles_of(c), _SEGT)],
            bufs.at[b, pl.ds(0, _SEGT)], in_sems.at[b])

    def out_copy(c, b):
        return pltpu.make_async_copy(
            bufs.at[b, pl.ds(0, _SEGT)],
            o_ref.at[pl.ds(tiles_of(c), _SEGT)], out_sems.at[b])

    for c in range(min(_PRIME, n)):
        in_copy(c, c % _RING).start()
    for c in range(n):
        b = c % _RING
        if c >= _OUTLAG:
            out_copy(c - _OUTLAG, (c - _OUTLAG) % _RING).wait()
        if c + _PRIME < n:
            in_copy(c + _PRIME, (c + _PRIME) % _RING).start()
        in_copy(c, b).wait()
        seg = c % _NSEG
        k = _SEG2PIECE.get(seg)
        if k is not None:
            _, t0, nt, _ = _PIECES[k]
            add = pltpu.make_async_copy(
                cr_ref.at[k].at[idx_buf.at[pl.ds(0, nt)]],
                bufs.at[b, pl.ds(t0, nt)], add_sem)
            add.start(add=True)
            add.wait()
        out_copy(c, b).start()
    for c in range(max(0, n - _OUTLAG), n):
        out_copy(c, c % _RING).wait()


def kernel(x, patch):
    B = x.shape[0]
    cruns = pl.pallas_call(
        _cruns_kernel,
        out_shape=jax.ShapeDtypeStruct((_NP, _CRT * _TILE), x.dtype),
    )(patch)
    cr4 = cruns.reshape(_NP, _CRT, 8, 128)
    x4 = x.reshape(B * _IMGT, 8, 128)
    arange = jnp.arange(16, dtype=jnp.int32)

    mesh = plsc.VectorSubcoreMesh(core_axis_name="c", subcore_axis_name="s")
    out = pl.kernel(
        _sc_add_body,
        out_type=jax.ShapeDtypeStruct((B * _IMGT, 8, 128), x.dtype),
        mesh=mesh,
        scratch_types=[
            pltpu.VMEM((_RING, _BUFT, 8, 128), x.dtype),
            pltpu.VMEM((16,), jnp.int32),
            pltpu.SemaphoreType.DMA,
            pltpu.SemaphoreType.DMA((_RING,)),
            pltpu.SemaphoreType.DMA((_RING,)),
            pltpu.SemaphoreType.DMA,
        ],
        compiler_params=pltpu.CompilerParams(use_tc_tiling_on_sc=True),
    )(x4, cr4, arange)
    return out.reshape(x.shape)


# speedup vs baseline: 1.0337x; 1.0294x over previous
import numpy as np
import jax
import jax.numpy as jnp
from jax import lax
from jax.experimental import pallas as pl
from jax.experimental.pallas import tpu as pltpu
from jax.experimental.pallas import tpu_sc as plsc

_ISIZE = 224
_PSIZE = 30
_rng = np.random.RandomState(0)
_X = int(_rng.randint(0, _ISIZE - _PSIZE))
_Y = int(_rng.randint(0, _ISIZE - _PSIZE))

_IMG = 3 * _ISIZE * _ISIZE
_NSEG = 7
_SEG = _IMG // _NSEG

_vregs = set()
_prows = []
for _c in range(3):
    for _r in range(_PSIZE):
        _f0 = _c * _ISIZE * _ISIZE + (_X + _r) * _ISIZE + _Y
        _prows.append((_c, _r, _f0))
        for _v in range(_f0 // 16, (_f0 + _PSIZE - 1) // 16 + 1):
            _vregs.add(_v)
_VS = sorted(_vregs)
_NV = len(_VS)
_V2C = {v: i for i, v in enumerate(_VS)}

_SEG_OPS = {}
for _v in _VS:
    _seg = (_v * 16) // _SEG
    assert (_v * 16 + 15) // _SEG == _seg
    _SEG_OPS.setdefault(_seg, []).append((_v * 16 - _seg * _SEG, _V2C[_v] * 16))

_RING = 5
_PRIME = 3
_OUTLAG = 2


def _ctab_kernel(p_ref, ct_ref):
    ct_ref[...] = jnp.zeros_like(ct_ref)
    for c, r, f0 in _prows:
        pos = _V2C[f0 // 16] * 16 + (f0 % 16)
        ct_ref[pos:pos + _PSIZE] = p_ref[0, c, r, :]


def _sc_add_body(x_ref, ct_ref, o_ref, bufs, ctab, csem,
                 in_sems, out_sems):
    info = plsc.get_sparse_core_info()
    nw = info.num_cores * info.num_subcores
    wid = lax.axis_index("s") * info.num_cores + lax.axis_index("c")
    n_imgs = x_ref.shape[0] // _IMG // nw
    base = wid * (n_imgs * _IMG)

    pltpu.make_async_copy(ct_ref, ctab, csem).start()
    pltpu.make_async_copy(ct_ref, ctab, csem).wait()

    n = n_imgs * _NSEG

    def off_of(c):
        img, seg = divmod(c, _NSEG)
        return base + img * _IMG + seg * _SEG

    def in_copy(c, b):
        return pltpu.make_async_copy(
            x_ref.at[pl.ds(off_of(c), _SEG)],
            bufs.at[pl.ds(b * _SEG, _SEG)], in_sems.at[b])

    def out_copy(c, b):
        return pltpu.make_async_copy(
            bufs.at[pl.ds(b * _SEG, _SEG)],
            o_ref.at[pl.ds(off_of(c), _SEG)], out_sems.at[b])

    for c in range(min(_PRIME, n)):
        in_copy(c, c % _RING).start()
    for c in range(n):
        b = c % _RING
        if c >= _OUTLAG:
            out_copy(c - _OUTLAG, (c - _OUTLAG) % _RING).wait()
        if c + _PRIME < n:
            in_copy(c + _PRIME, (c + _PRIME) % _RING).start()
        in_copy(c, b).wait()
        for loc, cpos in _SEG_OPS.get(c % _NSEG, ()):
            val = ctab[pl.ds(cpos, 16)]
            plsc.addupdate(bufs.at[pl.ds(b * _SEG + loc, 16)], val)
        out_copy(c, b).start()
    for c in range(max(0, n - _OUTLAG), n):
        out_copy(c, c % _RING).wait()


def kernel(x, patch):
    B = x.shape[0]
    ctab = pl.pallas_call(
        _ctab_kernel,
        out_shape=jax.ShapeDtypeStruct((_NV * 16,), x.dtype),
    )(patch)
    x1 = x.reshape(B * _IMG)

    mesh = plsc.VectorSubcoreMesh(core_axis_name="c", subcore_axis_name="s")
    out = pl.kernel(
        _sc_add_body,
        out_type=jax.ShapeDtypeStruct((B * _IMG,), x.dtype),
        mesh=mesh,
        scratch_types=[
            pltpu.VMEM((_RING * _SEG,), x.dtype),
            pltpu.VMEM((_NV * 16,), x.dtype),
            pltpu.SemaphoreType.DMA,
            pltpu.SemaphoreType.DMA((_RING,)),
            pltpu.SemaphoreType.DMA((_RING,)),
        ],
    )(x1, ctab)
    return out.reshape(x.shape)


# speedup vs baseline: 1.8707x; 1.8097x over previous
import numpy as np
import jax
import jax.numpy as jnp
from jax import lax
from jax.experimental import pallas as pl
from jax.experimental.pallas import tpu as pltpu
from jax.experimental.pallas import tpu_sc as plsc

_ISIZE = 224
_PSIZE = 30
_rng = np.random.RandomState(0)
_X = int(_rng.randint(0, _ISIZE - _PSIZE))
_Y = int(_rng.randint(0, _ISIZE - _PSIZE))

_IMG = 3 * _ISIZE * _ISIZE
_NSEG = 7
_SEG = _IMG // _NSEG
_ROWS = _IMG // 128

_prows = []
for _c in range(3):
    for _r in range(_PSIZE):
        _f0 = _c * _ISIZE * _ISIZE + (_X + _r) * _ISIZE + _Y
        assert (_f0 + _PSIZE - 1) // _SEG == _f0 // _SEG
        _prows.append((len(_prows), _c, _r, _f0, _f0 % 16))
_SEG_ROWS = {}
for _i, _c, _r, _f0, _ph in _prows:
    _SEG_ROWS.setdefault(_f0 // _SEG, []).append((_i, _f0))
_PHASES = np.array([p for _, _, _, _, p in _prows])


def _sc_canvas_body(prow_ref, z_ref, cv_ref, buf, prow_buf, sems):
    info = plsc.get_sparse_core_info()
    wid = lax.axis_index("s") * info.num_cores + lax.axis_index("c")

    @pl.when(wid < _NSEG)
    def _():
        pltpu.make_async_copy(prow_ref, prow_buf, sems.at[0]).start()
        pltpu.make_async_copy(prow_ref, prow_buf, sems.at[0]).wait()

    for w in range(_NSEG):
        @pl.when(wid == w)
        def _(w=w):
            pltpu.make_async_copy(z_ref, buf, sems.at[1]).start()
            pltpu.make_async_copy(z_ref, buf, sems.at[1]).wait()
            for i, f0 in _SEG_ROWS.get(w, ()):
                dst16 = (f0 - f0 % 16) - w * _SEG
                for j in range(3):
                    buf[pl.ds(dst16 + 16 * j, 16)] = (
                        prow_buf[pl.ds(i * 48 + 16 * j, 16)])
            pltpu.make_async_copy(
                buf, cv_ref.at[pl.ds(w * _SEG, _SEG)], sems.at[2]).start()
            pltpu.make_async_copy(
                buf, cv_ref.at[pl.ds(w * _SEG, _SEG)], sems.at[2]).wait()


def _add_kernel(x_ref, c_ref, o_ref):
    o_ref[...] = x_ref[...] + c_ref[...]


def kernel(x, patch):
    B = x.shape[0]
    rows = patch.reshape(len(_prows), _PSIZE)
    prow = jnp.zeros((len(_prows), 48), dtype=x.dtype)
    prow = prow.at[np.arange(len(_prows))[:, None],
                   _PHASES[:, None] + np.arange(_PSIZE)[None, :]].set(rows)
    prow = prow.reshape(-1)
    zseg = jnp.zeros((_SEG,), dtype=x.dtype)

    mesh = plsc.VectorSubcoreMesh(core_axis_name="c", subcore_axis_name="s")
    canvas = pl.kernel(
        _sc_canvas_body,
        out_type=jax.ShapeDtypeStruct((_IMG,), x.dtype),
        mesh=mesh,
        scratch_types=[
            pltpu.VMEM((_SEG,), x.dtype),
            pltpu.VMEM((len(_prows) * 48,), x.dtype),
            pltpu.SemaphoreType.DMA((3,)),
        ],
    )(prow, zseg)

    x2 = x.reshape(B, _ROWS, 128)
    c2 = canvas.reshape(1, _ROWS, 128)
    Bb = 4
    out = pl.pallas_call(
        _add_kernel,
        grid=(B // Bb,),
        in_specs=[
            pl.BlockSpec((Bb, _ROWS, 128), lambda i: (i, 0, 0)),
            pl.BlockSpec((1, _ROWS, 128), lambda i: (0, 0, 0)),
        ],
        out_specs=pl.BlockSpec((Bb, _ROWS, 128), lambda i: (i, 0, 0)),
        out_shape=jax.ShapeDtypeStruct((B, _ROWS, 128), x.dtype),
    )(x2, c2)
    return out.reshape(x.shape)
